# Initial kernel scaffold; baseline (speedup 1.0000x reference)
import functools

import jax
import jax.numpy as jnp
from jax import lax
from jax.experimental import pallas as pl
from jax.experimental.pallas import tpu as pltpu
from jax.experimental.pallas import tpu_sc as plsc

N_NODES = 10000
N_EDGES = 160000
D = 256
HF = 128
NUM_GRAPHS = 64
OUT_DIM = 128

NP = 10240
EP = 163840
CHUNK = 128
NW = 32
ROWS_PER_W = EP // (NW * CHUNK)
STRIPE = NP // 16

R = 512
NB = NP // R


def _make_sc_layer(with_deg: bool):
    mesh = plsc.VectorSubcoreMesh(core_axis_name="c", subcore_axis_name="s")

    out_type = [jax.ShapeDtypeStruct((2, NP, HF), jnp.float32)]
    if with_deg:
        out_type.append(jax.ShapeDtypeStruct((NP, 8), jnp.float32))

    scratch = [
        pltpu.VMEM((ROWS_PER_W, CHUNK), jnp.int32),
        pltpu.VMEM((ROWS_PER_W, CHUNK), jnp.int32),
        pltpu.VMEM((CHUNK, HF), jnp.float32),
        pltpu.VMEM((STRIPE, HF), jnp.float32),
        pltpu.VMEM_SHARED((NP, HF), jnp.float32),
        pltpu.SemaphoreType.DMA,
    ]
    if with_deg:
        scratch += [
            pltpu.VMEM((CHUNK, 8), jnp.float32),
            pltpu.VMEM((STRIPE, 8), jnp.float32),
            pltpu.VMEM_SHARED((NP, 8), jnp.float32),
        ]

    def body(x0, x1, srcs, dsts, zrows, zrows8, ones8, *rest):
        if with_deg:
            (agg_out, deg_out, idxs, idxd, rows, stage, agg_sh, sem,
             onesv, stage8, deg_sh) = rest
        else:
            (agg_out, idxs, idxd, rows, stage, agg_sh, sem) = rest
        cid = lax.axis_index("c")
        sid = lax.axis_index("s")
        wid = sid * 2 + cid

        pltpu.sync_copy(zrows, stage)
        pltpu.sync_copy(stage, agg_sh.at[pl.ds(sid * STRIPE, STRIPE)])
        if with_deg:
            @pl.when(cid == 0)
            def _():
                pltpu.sync_copy(zrows8, stage8)
                pltpu.sync_copy(stage8, deg_sh.at[pl.ds(sid * STRIPE, STRIPE)])
                pltpu.sync_copy(ones8, onesv)
        plsc.subcore_barrier()

        pltpu.sync_copy(srcs.at[pl.ds(wid * ROWS_PER_W, ROWS_PER_W)], idxs)
        pltpu.sync_copy(dsts.at[pl.ds(wid * ROWS_PER_W, ROWS_PER_W)], idxd)

        def run_edges(table, count_deg):
            def step(j, carry):
                pltpu.async_copy(table.at[idxs.at[j]], rows, sem).wait()
                pltpu.sync_copy(rows, agg_sh.at[idxd.at[j]], add=True)
                if count_deg:
                    pltpu.sync_copy(onesv, deg_sh.at[idxd.at[j]], add=True)
                return carry
            lax.fori_loop(0, ROWS_PER_W, step, 0)

        @pl.when(cid == 0)
        def _():
            run_edges(x0, with_deg)

        @pl.when(cid == 1)
        def _():
            run_edges(x1, False)

        plsc.subcore_barrier()

        pltpu.sync_copy(agg_sh.at[pl.ds(sid * STRIPE, STRIPE)], stage)
        pltpu.sync_copy(stage, agg_out.at[cid, pl.ds(sid * STRIPE, STRIPE)])
        if with_deg:
            @pl.when(cid == 0)
            def _():
                pltpu.sync_copy(deg_sh.at[pl.ds(sid * STRIPE, STRIPE)], stage8)
                pltpu.sync_copy(stage8, deg_out.at[pl.ds(sid * STRIPE, STRIPE)])

    return pl.kernel(body, out_type=tuple(out_type), mesh=mesh,
                     scratch_types=scratch)


_sc_layer_deg = _make_sc_layer(True)
_sc_layer = _make_sc_layer(False)


def _dense_body(x0r, x1r, aggr, degr, wrr, wnr, br, gr, ber, o0r, o1r):
    x = jnp.concatenate([x0r[...], x1r[...]], axis=1)
    a = jnp.concatenate([aggr[0], aggr[1]], axis=1)
    dg = jnp.maximum(degr[:, 0:1], 1.0)
    a = a / dg
    t = (jnp.dot(x, wrr[...], preferred_element_type=jnp.float32)
         + jnp.dot(a, wnr[...], preferred_element_type=jnp.float32)
         + br[...])
    mu = jnp.mean(t, axis=1, keepdims=True)
    var = jnp.mean((t - mu) ** 2, axis=1, keepdims=True)
    y = (t - mu) * lax.rsqrt(var + 1e-5) * gr[...] + ber[...]
    h = jnp.maximum(y, 0.0) + x
    o0r[...] = h[:, :HF]
    o1r[...] = h[:, HF:]


_dense_layer = pl.pallas_call(
    _dense_body,
    grid=(NB,),
    in_specs=[
        pl.BlockSpec((R, HF), lambda i: (i, 0)),
        pl.BlockSpec((R, HF), lambda i: (i, 0)),
        pl.BlockSpec((2, R, HF), lambda i: (0, i, 0)),
        pl.BlockSpec((R, 8), lambda i: (i, 0)),
        pl.BlockSpec((D, D), lambda i: (0, 0)),
        pl.BlockSpec((D, D), lambda i: (0, 0)),
        pl.BlockSpec((1, D), lambda i: (0, 0)),
        pl.BlockSpec((1, D), lambda i: (0, 0)),
        pl.BlockSpec((1, D), lambda i: (0, 0)),
    ],
    out_specs=[
        pl.BlockSpec((R, HF), lambda i: (i, 0)),
        pl.BlockSpec((R, HF), lambda i: (i, 0)),
    ],
    out_shape=[
        jax.ShapeDtypeStruct((NP, HF), jnp.float32),
        jax.ShapeDtypeStruct((NP, HF), jnp.float32),
    ],
)


def _readout_body(h0r, h1r, btr, wh1r, bh1r, wh2r, bh2r, wh3r, bh3r,
                  outr, s_ref, c_ref):
    i = pl.program_id(0)

    @pl.when(i == 0)
    def _():
        s_ref[...] = jnp.zeros_like(s_ref)
        c_ref[...] = jnp.zeros_like(c_ref)

    h = jnp.concatenate([h0r[...], h1r[...]], axis=1)
    ids = lax.broadcasted_iota(jnp.int32, (R, NUM_GRAPHS), 1)
    oh = (btr[...] == ids).astype(jnp.float32)
    s_ref[...] += lax.dot_general(oh, h, (((0,), (0,)), ((), ())),
                                  preferred_element_type=jnp.float32)
    c_ref[...] += lax.dot_general(oh, jnp.ones((R, HF), jnp.float32),
                                  (((0,), (0,)), ((), ())),
                                  preferred_element_type=jnp.float32)

    @pl.when(i == NB - 1)
    def _():
        hg = s_ref[...] / jnp.maximum(c_ref[:, 0:1], 1.0)
        u = jnp.maximum(jnp.dot(hg, wh1r[...],
                                preferred_element_type=jnp.float32)
                        + bh1r[...], 0.0)
        v = jnp.maximum(jnp.dot(u, wh2r[...],
                                preferred_element_type=jnp.float32)
                        + bh2r[...], 0.0)
        outr[...] = jnp.dot(v, wh3r[...],
                            preferred_element_type=jnp.float32) + bh3r[...]


_readout = pl.pallas_call(
    _readout_body,
    grid=(NB,),
    in_specs=[
        pl.BlockSpec((R, HF), lambda i: (i, 0)),
        pl.BlockSpec((R, HF), lambda i: (i, 0)),
        pl.BlockSpec((R, 1), lambda i: (i, 0)),
        pl.BlockSpec((D, 128), lambda i: (0, 0)),
        pl.BlockSpec((1, 128), lambda i: (0, 0)),
        pl.BlockSpec((128, 64), lambda i: (0, 0)),
        pl.BlockSpec((1, 64), lambda i: (0, 0)),
        pl.BlockSpec((64, 128), lambda i: (0, 0)),
        pl.BlockSpec((1, 128), lambda i: (0, 0)),
    ],
    out_specs=pl.BlockSpec((NUM_GRAPHS, OUT_DIM), lambda i: (0, 0)),
    out_shape=jax.ShapeDtypeStruct((NUM_GRAPHS, OUT_DIM), jnp.float32),
    scratch_shapes=[
        pltpu.VMEM((NUM_GRAPHS, D), jnp.float32),
        pltpu.VMEM((NUM_GRAPHS, HF), jnp.float32),
    ],
)


def kernel(x, edge_index, batch, Wr0, Wn0, b0, g0, be0, Wr1, Wn1, b1, g1,
           be1, Wr2, Wn2, b2, g2, be2, Wh1, bh1, Wh2, bh2, Wh3, bh3):
    f32 = jnp.float32
    x = x.astype(f32)

    pad_n = NP - N_NODES
    h0 = jnp.pad(x[:, :HF], ((0, pad_n), (0, 0)))
    h1 = jnp.pad(x[:, HF:], ((0, pad_n), (0, 0)))

    pad_e = EP - N_EDGES
    fill = N_NODES + (jnp.arange(pad_e, dtype=jnp.int32) % pad_n)
    src = jnp.concatenate([edge_index[0].astype(jnp.int32), fill])
    dst = jnp.concatenate([edge_index[1].astype(jnp.int32), fill])
    src = src.reshape(EP // CHUNK, CHUNK)
    dst = dst.reshape(EP // CHUNK, CHUNK)

    zrows = jnp.zeros((STRIPE, HF), f32)
    zrows8 = jnp.zeros((STRIPE, 8), f32)
    ones8 = jnp.ones((CHUNK, 8), f32)

    bt = jnp.pad(batch.astype(jnp.int32), (0, pad_n),
                 constant_values=NUM_GRAPHS).reshape(NP, 1)

    agg, deg = _sc_layer_deg(h0, h1, src, dst, zrows, zrows8, ones8)
    b0r = b0.reshape(1, D); g0r = g0.reshape(1, D); be0r = be0.reshape(1, D)
    h0, h1 = _dense_layer(h0, h1, agg, deg, Wr0, Wn0, b0r, g0r, be0r)

    agg = _sc_layer(h0, h1, src, dst, zrows, zrows8, ones8)
    b1r = b1.reshape(1, D); g1r = g1.reshape(1, D); be1r = be1.reshape(1, D)
    h0, h1 = _dense_layer(h0, h1, agg, deg, Wr1, Wn1, b1r, g1r, be1r)

    agg = _sc_layer(h0, h1, src, dst, zrows, zrows8, ones8)
    b2r = b2.reshape(1, D); g2r = g2.reshape(1, D); be2r = be2.reshape(1, D)
    h0, h1 = _dense_layer(h0, h1, agg, deg, Wr2, Wn2, b2r, g2r, be2r)

    out = _readout(h0, h1, bt, Wh1, bh1.reshape(1, 128), Wh2,
                   bh2.reshape(1, 64), Wh3, bh3.reshape(1, 128))
    return out


# speedup vs baseline: 1.7599x; 1.7599x over previous
import jax
import jax.numpy as jnp
from jax import lax
from jax.experimental import pallas as pl
from jax.experimental.pallas import tpu as pltpu
from jax.experimental.pallas import tpu_sc as plsc

N_NODES = 10000
N_EDGES = 160000
D = 256
HF = 128
NUM_GRAPHS = 64
OUT_DIM = 128

NP = 10240
NT = 16
TR = NP // NT
CH = 128
SLAB_CH = 1280
NPIECE = TR // CH

R = 512
NB = NP // R


_SC_MESH = plsc.VectorSubcoreMesh(core_axis_name="c", subcore_axis_name="s",
                                  num_cores=2, num_subcores=16)

_SC_SCRATCH = [
    pltpu.VMEM((CH,), jnp.int32),
    pltpu.VMEM((CH,), jnp.int32),
    pltpu.VMEM((CH, HF), jnp.float32),
    pltpu.VMEM((NT, 16), jnp.int32),
    pltpu.VMEM((NT, 16), jnp.int32),
    pltpu.VMEM_SHARED((NP, HF), jnp.float32),
    pltpu.SemaphoreType.DMA,
]


def _sc_agg_body(x0, x1, sslab, dslab, nch, cbase, zrows,
                 agg_out, srccur, dstcur, rows, nch_v, cbase_v, acc_sh, sem):
    cid = lax.axis_index("c")
    sid = lax.axis_index("s")

    pltpu.sync_copy(zrows, rows)
    for p in range(NPIECE):
        pltpu.sync_copy(rows, acc_sh.at[pl.ds(sid * TR + p * CH, CH)])
    pltpu.sync_copy(nch, nch_v)
    pltpu.sync_copy(cbase, cbase_v)
    plsc.subcore_barrier()

    n = nch_v[sid][0]
    base = cbase_v[sid][0]

    def run_edges(table):
        @pl.loop(0, n)
        def _(j):
            pltpu.sync_copy(sslab.at[base + j], srccur)
            pltpu.sync_copy(dslab.at[base + j], dstcur)
            pltpu.async_copy(table.at[srccur], rows, sem).wait()
            pltpu.sync_copy(rows, acc_sh.at[dstcur], add=True)

    @pl.when(cid == 0)
    def _():
        run_edges(x0)

    @pl.when(cid == 1)
    def _():
        run_edges(x1)

    plsc.subcore_barrier()

    for p in range(NPIECE):
        off = sid * TR + p * CH
        pltpu.sync_copy(acc_sh.at[pl.ds(off, CH)], rows)
        pltpu.sync_copy(rows, agg_out.at[cid, pl.ds(off, CH)])


_sc_agg = pl.kernel(
    _sc_agg_body,
    out_type=jax.ShapeDtypeStruct((2, NP, HF), jnp.float32),
    mesh=_SC_MESH,
    scratch_types=_SC_SCRATCH,
)


def _sc_deg_body(ones, dslab, nch, cbase, zrows,
                 deg_out, srccur, dstcur, rows, nch_v, cbase_v, acc_sh, sem):
    cid = lax.axis_index("c")
    sid = lax.axis_index("s")

    pltpu.sync_copy(zrows, rows)
    for p in range(NPIECE):
        pltpu.sync_copy(rows, acc_sh.at[pl.ds(sid * TR + p * CH, CH)])
    pltpu.sync_copy(nch, nch_v)
    pltpu.sync_copy(cbase, cbase_v)
    pltpu.sync_copy(ones, rows)
    plsc.subcore_barrier()

    n = nch_v[sid][0]
    base = cbase_v[sid][0]
    half = n // 2
    lo = cid * half
    hi = half + cid * (n - half)

    @pl.loop(lo, hi)
    def _(j):
        pltpu.sync_copy(dslab.at[base + j], dstcur)
        pltpu.sync_copy(rows, acc_sh.at[dstcur], add=True)

    plsc.subcore_barrier()
    for p in range(NPIECE):
        off = sid * TR + p * CH
        pltpu.sync_copy(acc_sh.at[pl.ds(off, CH)], rows)
        pltpu.sync_copy(rows, deg_out.at[cid, pl.ds(off, CH)])


_sc_deg = pl.kernel(
    _sc_deg_body,
    out_type=jax.ShapeDtypeStruct((2, NP, HF), jnp.float32),
    mesh=_SC_MESH,
    scratch_types=_SC_SCRATCH,
)


def _dense_body(x0r, x1r, aggr, dgr, wrr, wnr, br, gr, ber, o0r, o1r):
    x = jnp.concatenate([x0r[...], x1r[...]], axis=1)
    a = jnp.concatenate([aggr[0], aggr[1]], axis=1)
    dg = jnp.maximum(dgr[0, :, 0:1] + dgr[1, :, 0:1], 1.0)
    a = a / dg
    t = (jnp.dot(x, wrr[...], preferred_element_type=jnp.float32)
         + jnp.dot(a, wnr[...], preferred_element_type=jnp.float32)
         + br[...])
    mu = jnp.mean(t, axis=1, keepdims=True)
    var = jnp.mean((t - mu) ** 2, axis=1, keepdims=True)
    y = (t - mu) * lax.rsqrt(var + 1e-5) * gr[...] + ber[...]
    h = jnp.maximum(y, 0.0) + x
    o0r[...] = h[:, :HF]
    o1r[...] = h[:, HF:]


_dense_layer = pl.pallas_call(
    _dense_body,
    grid=(NB,),
    in_specs=[
        pl.BlockSpec((R, HF), lambda i: (i, 0)),
        pl.BlockSpec((R, HF), lambda i: (i, 0)),
        pl.BlockSpec((2, R, HF), lambda i: (0, i, 0)),
        pl.BlockSpec((2, R, HF), lambda i: (0, i, 0)),
        pl.BlockSpec((D, D), lambda i: (0, 0)),
        pl.BlockSpec((D, D), lambda i: (0, 0)),
        pl.BlockSpec((1, D), lambda i: (0, 0)),
        pl.BlockSpec((1, D), lambda i: (0, 0)),
        pl.BlockSpec((1, D), lambda i: (0, 0)),
    ],
    out_specs=[
        pl.BlockSpec((R, HF), lambda i: (i, 0)),
        pl.BlockSpec((R, HF), lambda i: (i, 0)),
    ],
    out_shape=[
        jax.ShapeDtypeStruct((NP, HF), jnp.float32),
        jax.ShapeDtypeStruct((NP, HF), jnp.float32),
    ],
)


def _readout_body(h0r, h1r, btr, wh1r, bh1r, wh2r, bh2r, wh3r, bh3r,
                  outr, s_ref, c_ref):
    i = pl.program_id(0)

    @pl.when(i == 0)
    def _():
        s_ref[...] = jnp.zeros_like(s_ref)
        c_ref[...] = jnp.zeros_like(c_ref)

    h = jnp.concatenate([h0r[...], h1r[...]], axis=1)
    ids = lax.broadcasted_iota(jnp.int32, (R, NUM_GRAPHS), 1)
    oh = (btr[...] == ids).astype(jnp.float32)
    s_ref[...] += lax.dot_general(oh, h, (((0,), (0,)), ((), ())),
                                  preferred_element_type=jnp.float32)
    c_ref[...] += lax.dot_general(oh, jnp.ones((R, HF), jnp.float32),
                                  (((0,), (0,)), ((), ())),
                                  preferred_element_type=jnp.float32)

    @pl.when(i == NB - 1)
    def _():
        hg = s_ref[...] / jnp.maximum(c_ref[:, 0:1], 1.0)
        u = jnp.maximum(jnp.dot(hg, wh1r[...],
                                preferred_element_type=jnp.float32)
                        + bh1r[...], 0.0)
        v = jnp.maximum(jnp.dot(u, wh2r[...],
                                preferred_element_type=jnp.float32)
                        + bh2r[...], 0.0)
        outr[...] = jnp.dot(v, wh3r[...],
                            preferred_element_type=jnp.float32) + bh3r[...]


_readout = pl.pallas_call(
    _readout_body,
    grid=(NB,),
    in_specs=[
        pl.BlockSpec((R, HF), lambda i: (i, 0)),
        pl.BlockSpec((R, HF), lambda i: (i, 0)),
        pl.BlockSpec((R, 1), lambda i: (i, 0)),
        pl.BlockSpec((D, 128), lambda i: (0, 0)),
        pl.BlockSpec((1, 128), lambda i: (0, 0)),
        pl.BlockSpec((128, 64), lambda i: (0, 0)),
        pl.BlockSpec((1, 64), lambda i: (0, 0)),
        pl.BlockSpec((64, 128), lambda i: (0, 0)),
        pl.BlockSpec((1, 128), lambda i: (0, 0)),
    ],
    out_specs=pl.BlockSpec((NUM_GRAPHS, OUT_DIM), lambda i: (0, 0)),
    out_shape=jax.ShapeDtypeStruct((NUM_GRAPHS, OUT_DIM), jnp.float32),
    scratch_shapes=[
        pltpu.VMEM((NUM_GRAPHS, D), jnp.float32),
        pltpu.VMEM((NUM_GRAPHS, HF), jnp.float32),
    ],
)


def kernel(x, edge_index, batch, Wr0, Wn0, b0, g0, be0, Wr1, Wn1, b1, g1,
           be1, Wr2, Wn2, b2, g2, be2, Wh1, bh1, Wh2, bh2, Wh3, bh3):
    f32 = jnp.float32
    i32 = jnp.int32
    x = x.astype(f32)

    pad_n = NP - N_NODES
    h0 = jnp.pad(x[:, :HF], ((0, pad_n), (0, 0)))
    h1 = jnp.pad(x[:, HF:], ((0, pad_n), (0, 0)))

    src = edge_index[0].astype(i32)
    dst = edge_index[1].astype(i32)
    tl = dst // TR
    oh16 = (tl[:, None] == jnp.arange(NT, dtype=i32)[None, :]).astype(i32)
    ranks = jnp.cumsum(oh16, axis=0)
    rank = jnp.take_along_axis(ranks, tl[:, None], 1)[:, 0] - 1
    cnt = ranks[-1]
    nch = (cnt + CH - 1) // CH
    cbase = jnp.concatenate([jnp.zeros((1,), i32),
                             jnp.cumsum(nch)[:-1].astype(i32)])
    pos = jnp.take(cbase, tl) * CH + rank

    slot = jnp.arange(SLAB_CH * CH, dtype=i32)
    sslab = jnp.full((SLAB_CH * CH,), N_NODES, i32).at[pos].set(src)
    dslab = (N_NODES + slot % pad_n).at[pos].set(dst)
    sslab = sslab.reshape(SLAB_CH, CH)
    dslab = dslab.reshape(SLAB_CH, CH)
    nch = (nch.astype(i32)[:, None] * jnp.ones((1, 16), i32))
    cbase = (cbase.astype(i32)[:, None] * jnp.ones((1, 16), i32))

    zrows = jnp.zeros((CH, HF), f32)
    ones = jnp.ones((CH, HF), f32)

    bt = jnp.pad(batch.astype(i32), (0, pad_n),
                 constant_values=NUM_GRAPHS).reshape(NP, 1)

    deg = _sc_deg(ones, dslab, nch, cbase, zrows)

    agg = _sc_agg(h0, h1, sslab, dslab, nch, cbase, zrows)
    b0r = b0.reshape(1, D); g0r = g0.reshape(1, D); be0r = be0.reshape(1, D)
    h0, h1 = _dense_layer(h0, h1, agg, deg, Wr0, Wn0, b0r, g0r, be0r)

    agg = _sc_agg(h0, h1, sslab, dslab, nch, cbase, zrows)
    b1r = b1.reshape(1, D); g1r = g1.reshape(1, D); be1r = be1.reshape(1, D)
    h0, h1 = _dense_layer(h0, h1, agg, deg, Wr1, Wn1, b1r, g1r, be1r)

    agg = _sc_agg(h0, h1, sslab, dslab, nch, cbase, zrows)
    b2r = b2.reshape(1, D); g2r = g2.reshape(1, D); be2r = be2.reshape(1, D)
    h0, h1 = _dense_layer(h0, h1, agg, deg, Wr2, Wn2, b2r, g2r, be2r)

    out = _readout(h0, h1, bt, Wh1, bh1.reshape(1, 128), Wh2,
                   bh2.reshape(1, 64), Wh3, bh3.reshape(1, 128))
    return out


# speedup vs baseline: 1.8165x; 1.0322x over previous
import jax
import jax.numpy as jnp
from jax import lax
from jax.experimental import pallas as pl
from jax.experimental.pallas import tpu as pltpu
from jax.experimental.pallas import tpu_sc as plsc

N_NODES = 10000
N_EDGES = 160000
D = 256
HF = 128
NUM_GRAPHS = 64
OUT_DIM = 128

NP = 10240
NT = 16
TR = NP // NT
CH = 256
SLAB_CH = 656
_PIECES = ((0, 256), (256, 256), (512, 128))

R = 512
NB = NP // R


_SC_MESH = plsc.VectorSubcoreMesh(core_axis_name="c", subcore_axis_name="s",
                                  num_cores=2, num_subcores=16)

_SC_SCRATCH = [
    pltpu.VMEM((CH,), jnp.int32),
    pltpu.VMEM((CH,), jnp.int32),
    pltpu.VMEM((CH, HF), jnp.float32),
    pltpu.VMEM((NT, 16), jnp.int32),
    pltpu.VMEM((NT, 16), jnp.int32),
    pltpu.VMEM_SHARED((NP, HF), jnp.float32),
    pltpu.SemaphoreType.DMA,
]


def _sc_agg_body(x0, x1, sslab, dslab, nch, cbase, zrows,
                 agg_out, srccur, dstcur, rows, nch_v, cbase_v, acc_sh, sem):
    cid = lax.axis_index("c")
    sid = lax.axis_index("s")

    pltpu.sync_copy(zrows, rows)
    for off, sz in _PIECES:
        pltpu.sync_copy(rows.at[pl.ds(0, sz)],
                        acc_sh.at[pl.ds(sid * TR + off, sz)])
    pltpu.sync_copy(nch, nch_v)
    pltpu.sync_copy(cbase, cbase_v)
    plsc.subcore_barrier()

    n = nch_v[sid][0]
    base = cbase_v[sid][0]

    def run_edges(table):
        @pl.loop(0, n)
        def _(j):
            pltpu.sync_copy(sslab.at[base + j], srccur)
            pltpu.sync_copy(dslab.at[base + j], dstcur)
            pltpu.async_copy(table.at[srccur], rows, sem).wait()
            pltpu.sync_copy(rows, acc_sh.at[dstcur], add=True)

    @pl.when(cid == 0)
    def _():
        run_edges(x0)

    @pl.when(cid == 1)
    def _():
        run_edges(x1)

    plsc.subcore_barrier()

    for off, sz in _PIECES:
        o = sid * TR + off
        pltpu.sync_copy(acc_sh.at[pl.ds(o, sz)], rows.at[pl.ds(0, sz)])
        pltpu.sync_copy(rows.at[pl.ds(0, sz)], agg_out.at[cid, pl.ds(o, sz)])


_sc_agg = pl.kernel(
    _sc_agg_body,
    out_type=jax.ShapeDtypeStruct((2, NP, HF), jnp.float32),
    mesh=_SC_MESH,
    scratch_types=_SC_SCRATCH,
)


def _sc_deg_body(ones, dslab, nch, cbase, zrows,
                 deg_out, srccur, dstcur, rows, nch_v, cbase_v, acc_sh, sem):
    cid = lax.axis_index("c")
    sid = lax.axis_index("s")

    pltpu.sync_copy(zrows, rows)
    for off, sz in _PIECES:
        pltpu.sync_copy(rows.at[pl.ds(0, sz)],
                        acc_sh.at[pl.ds(sid * TR + off, sz)])
    pltpu.sync_copy(nch, nch_v)
    pltpu.sync_copy(cbase, cbase_v)
    pltpu.sync_copy(ones, rows)
    plsc.subcore_barrier()

    n = nch_v[sid][0]
    base = cbase_v[sid][0]
    half = n // 2
    lo = cid * half
    hi = half + cid * (n - half)

    @pl.loop(lo, hi)
    def _(j):
        pltpu.sync_copy(dslab.at[base + j], dstcur)
        pltpu.sync_copy(rows, acc_sh.at[dstcur], add=True)

    plsc.subcore_barrier()
    for off, sz in _PIECES:
        o = sid * TR + off
        pltpu.sync_copy(acc_sh.at[pl.ds(o, sz)], rows.at[pl.ds(0, sz)])
        pltpu.sync_copy(rows.at[pl.ds(0, sz)], deg_out.at[cid, pl.ds(o, sz)])


_sc_deg = pl.kernel(
    _sc_deg_body,
    out_type=jax.ShapeDtypeStruct((2, NP, HF), jnp.float32),
    mesh=_SC_MESH,
    scratch_types=_SC_SCRATCH,
)


def _dense_body(x0r, x1r, aggr, dgr, wrr, wnr, br, gr, ber, o0r, o1r):
    x = jnp.concatenate([x0r[...], x1r[...]], axis=1)
    a = jnp.concatenate([aggr[0], aggr[1]], axis=1)
    dg = jnp.maximum(dgr[0, :, 0:1] + dgr[1, :, 0:1], 1.0)
    a = a / dg
    t = (jnp.dot(x, wrr[...], preferred_element_type=jnp.float32)
         + jnp.dot(a, wnr[...], preferred_element_type=jnp.float32)
         + br[...])
    mu = jnp.mean(t, axis=1, keepdims=True)
    var = jnp.mean((t - mu) ** 2, axis=1, keepdims=True)
    y = (t - mu) * lax.rsqrt(var + 1e-5) * gr[...] + ber[...]
    h = jnp.maximum(y, 0.0) + x
    o0r[...] = h[:, :HF]
    o1r[...] = h[:, HF:]


_dense_layer = pl.pallas_call(
    _dense_body,
    grid=(NB,),
    in_specs=[
        pl.BlockSpec((R, HF), lambda i: (i, 0)),
        pl.BlockSpec((R, HF), lambda i: (i, 0)),
        pl.BlockSpec((2, R, HF), lambda i: (0, i, 0)),
        pl.BlockSpec((2, R, HF), lambda i: (0, i, 0)),
        pl.BlockSpec((D, D), lambda i: (0, 0)),
        pl.BlockSpec((D, D), lambda i: (0, 0)),
        pl.BlockSpec((1, D), lambda i: (0, 0)),
        pl.BlockSpec((1, D), lambda i: (0, 0)),
        pl.BlockSpec((1, D), lambda i: (0, 0)),
    ],
    out_specs=[
        pl.BlockSpec((R, HF), lambda i: (i, 0)),
        pl.BlockSpec((R, HF), lambda i: (i, 0)),
    ],
    out_shape=[
        jax.ShapeDtypeStruct((NP, HF), jnp.float32),
        jax.ShapeDtypeStruct((NP, HF), jnp.float32),
    ],
)


def _readout_body(h0r, h1r, btr, wh1r, bh1r, wh2r, bh2r, wh3r, bh3r,
                  outr, s_ref, c_ref):
    i = pl.program_id(0)

    @pl.when(i == 0)
    def _():
        s_ref[...] = jnp.zeros_like(s_ref)
        c_ref[...] = jnp.zeros_like(c_ref)

    h = jnp.concatenate([h0r[...], h1r[...]], axis=1)
    ids = lax.broadcasted_iota(jnp.int32, (R, NUM_GRAPHS), 1)
    oh = (btr[...] == ids).astype(jnp.float32)
    s_ref[...] += lax.dot_general(oh, h, (((0,), (0,)), ((), ())),
                                  preferred_element_type=jnp.float32)
    c_ref[...] += lax.dot_general(oh, jnp.ones((R, HF), jnp.float32),
                                  (((0,), (0,)), ((), ())),
                                  preferred_element_type=jnp.float32)

    @pl.when(i == NB - 1)
    def _():
        hg = s_ref[...] / jnp.maximum(c_ref[:, 0:1], 1.0)
        u = jnp.maximum(jnp.dot(hg, wh1r[...],
                                preferred_element_type=jnp.float32)
                        + bh1r[...], 0.0)
        v = jnp.maximum(jnp.dot(u, wh2r[...],
                                preferred_element_type=jnp.float32)
                        + bh2r[...], 0.0)
        outr[...] = jnp.dot(v, wh3r[...],
                            preferred_element_type=jnp.float32) + bh3r[...]


_readout = pl.pallas_call(
    _readout_body,
    grid=(NB,),
    in_specs=[
        pl.BlockSpec((R, HF), lambda i: (i, 0)),
        pl.BlockSpec((R, HF), lambda i: (i, 0)),
        pl.BlockSpec((R, 1), lambda i: (i, 0)),
        pl.BlockSpec((D, 128), lambda i: (0, 0)),
        pl.BlockSpec((1, 128), lambda i: (0, 0)),
        pl.BlockSpec((128, 64), lambda i: (0, 0)),
        pl.BlockSpec((1, 64), lambda i: (0, 0)),
        pl.BlockSpec((64, 128), lambda i: (0, 0)),
        pl.BlockSpec((1, 128), lambda i: (0, 0)),
    ],
    out_specs=pl.BlockSpec((NUM_GRAPHS, OUT_DIM), lambda i: (0, 0)),
    out_shape=jax.ShapeDtypeStruct((NUM_GRAPHS, OUT_DIM), jnp.float32),
    scratch_shapes=[
        pltpu.VMEM((NUM_GRAPHS, D), jnp.float32),
        pltpu.VMEM((NUM_GRAPHS, HF), jnp.float32),
    ],
)


def kernel(x, edge_index, batch, Wr0, Wn0, b0, g0, be0, Wr1, Wn1, b1, g1,
           be1, Wr2, Wn2, b2, g2, be2, Wh1, bh1, Wh2, bh2, Wh3, bh3):
    f32 = jnp.float32
    i32 = jnp.int32
    x = x.astype(f32)

    pad_n = NP - N_NODES
    h0 = jnp.pad(x[:, :HF], ((0, pad_n), (0, 0)))
    h1 = jnp.pad(x[:, HF:], ((0, pad_n), (0, 0)))

    src = edge_index[0].astype(i32)
    dst = edge_index[1].astype(i32)
    tl = dst // TR
    oh16 = (tl[:, None] == jnp.arange(NT, dtype=i32)[None, :]).astype(i32)
    ranks = jnp.cumsum(oh16, axis=0)
    rank = jnp.take_along_axis(ranks, tl[:, None], 1)[:, 0] - 1
    cnt = ranks[-1]
    nch = (cnt + CH - 1) // CH
    cbase = jnp.concatenate([jnp.zeros((1,), i32),
                             jnp.cumsum(nch)[:-1].astype(i32)])
    pos = jnp.take(cbase, tl) * CH + rank

    slot = jnp.arange(SLAB_CH * CH, dtype=i32)
    sslab = jnp.full((SLAB_CH * CH,), N_NODES, i32).at[pos].set(src)
    dslab = (N_NODES + slot % pad_n).at[pos].set(dst)
    sslab = sslab.reshape(SLAB_CH, CH)
    dslab = dslab.reshape(SLAB_CH, CH)
    nch = (nch.astype(i32)[:, None] * jnp.ones((1, 16), i32))
    cbase = (cbase.astype(i32)[:, None] * jnp.ones((1, 16), i32))

    zrows = jnp.zeros((CH, HF), f32)
    ones = jnp.ones((CH, HF), f32)

    bt = jnp.pad(batch.astype(i32), (0, pad_n),
                 constant_values=NUM_GRAPHS).reshape(NP, 1)

    deg = _sc_deg(ones, dslab, nch, cbase, zrows)

    agg = _sc_agg(h0, h1, sslab, dslab, nch, cbase, zrows)
    b0r = b0.reshape(1, D); g0r = g0.reshape(1, D); be0r = be0.reshape(1, D)
    h0, h1 = _dense_layer(h0, h1, agg, deg, Wr0, Wn0, b0r, g0r, be0r)

    agg = _sc_agg(h0, h1, sslab, dslab, nch, cbase, zrows)
    b1r = b1.reshape(1, D); g1r = g1.reshape(1, D); be1r = be1.reshape(1, D)
    h0, h1 = _dense_layer(h0, h1, agg, deg, Wr1, Wn1, b1r, g1r, be1r)

    agg = _sc_agg(h0, h1, sslab, dslab, nch, cbase, zrows)
    b2r = b2.reshape(1, D); g2r = g2.reshape(1, D); be2r = be2.reshape(1, D)
    h0, h1 = _dense_layer(h0, h1, agg, deg, Wr2, Wn2, b2r, g2r, be2r)

    out = _readout(h0, h1, bt, Wh1, bh1.reshape(1, 128), Wh2,
                   bh2.reshape(1, 64), Wh3, bh3.reshape(1, 128))
    return out


# speedup vs baseline: 1.8562x; 1.0218x over previous
import jax
import jax.numpy as jnp
from jax import lax
from jax.experimental import pallas as pl
from jax.experimental.pallas import tpu as pltpu
from jax.experimental.pallas import tpu_sc as plsc

N_NODES = 10000
N_EDGES = 160000
D = 256
HF = 128
NUM_GRAPHS = 64
OUT_DIM = 128

NP = 10240
NT = 16
TR = NP // NT
CH = 128
SLAB_CH = 1296
_PIECES = tuple((p * CH, CH) for p in range(TR // CH))

R = 512
NB = NP // R


_SC_MESH = plsc.VectorSubcoreMesh(core_axis_name="c", subcore_axis_name="s",
                                  num_cores=2, num_subcores=16)

_SC_SCRATCH = [
    pltpu.VMEM((CH,), jnp.int32),
    pltpu.VMEM((CH,), jnp.int32),
    pltpu.VMEM((CH,), jnp.int32),
    pltpu.VMEM((CH,), jnp.int32),
    pltpu.VMEM((CH, HF), jnp.float32),
    pltpu.VMEM((CH, HF), jnp.float32),
    pltpu.VMEM((NT, 16), jnp.int32),
    pltpu.VMEM((NT, 16), jnp.int32),
    pltpu.VMEM_SHARED((NP, HF), jnp.float32),
    pltpu.SemaphoreType.DMA,
    pltpu.SemaphoreType.DMA,
]


def _sc_agg_body(x0, x1, sslab, dslab, nch, cbase, zrows,
                 agg_out, srccur, dstcur, srccur2, dstcur2, rows, rows2,
                 nch_v, cbase_v, acc_sh, sem, sem2):
    cid = lax.axis_index("c")
    sid = lax.axis_index("s")

    pltpu.sync_copy(zrows, rows)
    for off, sz in _PIECES:
        pltpu.sync_copy(rows.at[pl.ds(0, sz)],
                        acc_sh.at[pl.ds(sid * TR + off, sz)])
    pltpu.sync_copy(nch, nch_v)
    pltpu.sync_copy(cbase, cbase_v)
    plsc.subcore_barrier()

    n = nch_v[sid][0]
    base = cbase_v[sid][0]

    def run_edges(table):
        @pl.loop(0, n // 2)
        def _(jj):
            j0 = base + 2 * jj
            pltpu.sync_copy(sslab.at[j0], srccur)
            pltpu.sync_copy(dslab.at[j0], dstcur)
            cp0 = pltpu.async_copy(table.at[srccur], rows, sem)
            pltpu.sync_copy(sslab.at[j0 + 1], srccur2)
            pltpu.sync_copy(dslab.at[j0 + 1], dstcur2)
            cp1 = pltpu.async_copy(table.at[srccur2], rows2, sem2)
            cp0.wait()
            pltpu.sync_copy(rows, acc_sh.at[dstcur], add=True)
            cp1.wait()
            pltpu.sync_copy(rows2, acc_sh.at[dstcur2], add=True)

    @pl.when(cid == 0)
    def _():
        run_edges(x0)

    @pl.when(cid == 1)
    def _():
        run_edges(x1)

    plsc.subcore_barrier()

    for off, sz in _PIECES:
        o = sid * TR + off
        pltpu.sync_copy(acc_sh.at[pl.ds(o, sz)], rows.at[pl.ds(0, sz)])
        pltpu.sync_copy(rows.at[pl.ds(0, sz)], agg_out.at[cid, pl.ds(o, sz)])


_sc_agg = pl.kernel(
    _sc_agg_body,
    out_type=jax.ShapeDtypeStruct((2, NP, HF), jnp.float32),
    mesh=_SC_MESH,
    scratch_types=_SC_SCRATCH,
)


def _sc_deg_body(ones, dslab, nch, cbase, zrows,
                 deg_out, srccur, dstcur, srccur2, dstcur2, rows, rows2,
                 nch_v, cbase_v, acc_sh, sem, sem2):
    cid = lax.axis_index("c")
    sid = lax.axis_index("s")

    pltpu.sync_copy(zrows, rows)
    for off, sz in _PIECES:
        pltpu.sync_copy(rows.at[pl.ds(0, sz)],
                        acc_sh.at[pl.ds(sid * TR + off, sz)])
    pltpu.sync_copy(nch, nch_v)
    pltpu.sync_copy(cbase, cbase_v)
    pltpu.sync_copy(ones, rows)
    plsc.subcore_barrier()

    n = nch_v[sid][0]
    base = cbase_v[sid][0]
    half = n // 2
    lo = cid * half
    hi = half + cid * (n - half)

    @pl.loop(lo, hi)
    def _(j):
        pltpu.sync_copy(dslab.at[base + j], dstcur)
        pltpu.sync_copy(rows, acc_sh.at[dstcur], add=True)

    plsc.subcore_barrier()
    for off, sz in _PIECES:
        o = sid * TR + off
        pltpu.sync_copy(acc_sh.at[pl.ds(o, sz)], rows.at[pl.ds(0, sz)])
        pltpu.sync_copy(rows.at[pl.ds(0, sz)], deg_out.at[cid, pl.ds(o, sz)])


_sc_deg = pl.kernel(
    _sc_deg_body,
    out_type=jax.ShapeDtypeStruct((2, NP, HF), jnp.float32),
    mesh=_SC_MESH,
    scratch_types=_SC_SCRATCH,
)


def _dense_body(x0r, x1r, aggr, dgr, wrr, wnr, br, gr, ber, o0r, o1r):
    x = jnp.concatenate([x0r[...], x1r[...]], axis=1)
    a = jnp.concatenate([aggr[0], aggr[1]], axis=1)
    dg = jnp.maximum(dgr[0, :, 0:1] + dgr[1, :, 0:1], 1.0)
    a = a / dg
    t = (jnp.dot(x, wrr[...], preferred_element_type=jnp.float32)
         + jnp.dot(a, wnr[...], preferred_element_type=jnp.float32)
         + br[...])
    mu = jnp.mean(t, axis=1, keepdims=True)
    var = jnp.mean((t - mu) ** 2, axis=1, keepdims=True)
    y = (t - mu) * lax.rsqrt(var + 1e-5) * gr[...] + ber[...]
    h = jnp.maximum(y, 0.0) + x
    o0r[...] = h[:, :HF]
    o1r[...] = h[:, HF:]


_dense_layer = pl.pallas_call(
    _dense_body,
    grid=(NB,),
    in_specs=[
        pl.BlockSpec((R, HF), lambda i: (i, 0)),
        pl.BlockSpec((R, HF), lambda i: (i, 0)),
        pl.BlockSpec((2, R, HF), lambda i: (0, i, 0)),
        pl.BlockSpec((2, R, HF), lambda i: (0, i, 0)),
        pl.BlockSpec((D, D), lambda i: (0, 0)),
        pl.BlockSpec((D, D), lambda i: (0, 0)),
        pl.BlockSpec((1, D), lambda i: (0, 0)),
        pl.BlockSpec((1, D), lambda i: (0, 0)),
        pl.BlockSpec((1, D), lambda i: (0, 0)),
    ],
    out_specs=[
        pl.BlockSpec((R, HF), lambda i: (i, 0)),
        pl.BlockSpec((R, HF), lambda i: (i, 0)),
    ],
    out_shape=[
        jax.ShapeDtypeStruct((NP, HF), jnp.float32),
        jax.ShapeDtypeStruct((NP, HF), jnp.float32),
    ],
)


def _readout_body(h0r, h1r, btr, wh1r, bh1r, wh2r, bh2r, wh3r, bh3r,
                  outr, s_ref, c_ref):
    i = pl.program_id(0)

    @pl.when(i == 0)
    def _():
        s_ref[...] = jnp.zeros_like(s_ref)
        c_ref[...] = jnp.zeros_like(c_ref)

    h = jnp.concatenate([h0r[...], h1r[...]], axis=1)
    ids = lax.broadcasted_iota(jnp.int32, (R, NUM_GRAPHS), 1)
    oh = (btr[...] == ids).astype(jnp.float32)
    s_ref[...] += lax.dot_general(oh, h, (((0,), (0,)), ((), ())),
                                  preferred_element_type=jnp.float32)
    c_ref[...] += lax.dot_general(oh, jnp.ones((R, HF), jnp.float32),
                                  (((0,), (0,)), ((), ())),
                                  preferred_element_type=jnp.float32)

    @pl.when(i == NB - 1)
    def _():
        hg = s_ref[...] / jnp.maximum(c_ref[:, 0:1], 1.0)
        u = jnp.maximum(jnp.dot(hg, wh1r[...],
                                preferred_element_type=jnp.float32)
                        + bh1r[...], 0.0)
        v = jnp.maximum(jnp.dot(u, wh2r[...],
                                preferred_element_type=jnp.float32)
                        + bh2r[...], 0.0)
        outr[...] = jnp.dot(v, wh3r[...],
                            preferred_element_type=jnp.float32) + bh3r[...]


_readout = pl.pallas_call(
    _readout_body,
    grid=(NB,),
    in_specs=[
        pl.BlockSpec((R, HF), lambda i: (i, 0)),
        pl.BlockSpec((R, HF), lambda i: (i, 0)),
        pl.BlockSpec((R, 1), lambda i: (i, 0)),
        pl.BlockSpec((D, 128), lambda i: (0, 0)),
        pl.BlockSpec((1, 128), lambda i: (0, 0)),
        pl.BlockSpec((128, 64), lambda i: (0, 0)),
        pl.BlockSpec((1, 64), lambda i: (0, 0)),
        pl.BlockSpec((64, 128), lambda i: (0, 0)),
        pl.BlockSpec((1, 128), lambda i: (0, 0)),
    ],
    out_specs=pl.BlockSpec((NUM_GRAPHS, OUT_DIM), lambda i: (0, 0)),
    out_shape=jax.ShapeDtypeStruct((NUM_GRAPHS, OUT_DIM), jnp.float32),
    scratch_shapes=[
        pltpu.VMEM((NUM_GRAPHS, D), jnp.float32),
        pltpu.VMEM((NUM_GRAPHS, HF), jnp.float32),
    ],
)


def kernel(x, edge_index, batch, Wr0, Wn0, b0, g0, be0, Wr1, Wn1, b1, g1,
           be1, Wr2, Wn2, b2, g2, be2, Wh1, bh1, Wh2, bh2, Wh3, bh3):
    f32 = jnp.float32
    i32 = jnp.int32
    x = x.astype(f32)

    pad_n = NP - N_NODES
    h0 = jnp.pad(x[:, :HF], ((0, pad_n), (0, 0)))
    h1 = jnp.pad(x[:, HF:], ((0, pad_n), (0, 0)))

    src = edge_index[0].astype(i32)
    dst = edge_index[1].astype(i32)
    tl = dst // TR
    oh16 = (tl[:, None] == jnp.arange(NT, dtype=i32)[None, :]).astype(i32)
    ranks = jnp.cumsum(oh16, axis=0)
    rank = jnp.take_along_axis(ranks, tl[:, None], 1)[:, 0] - 1
    cnt = ranks[-1]
    nch = (cnt + CH - 1) // CH
    nch = ((nch + 1) // 2) * 2
    cbase = jnp.concatenate([jnp.zeros((1,), i32),
                             jnp.cumsum(nch)[:-1].astype(i32)])
    pos = jnp.take(cbase, tl) * CH + rank

    slot = jnp.arange(SLAB_CH * CH, dtype=i32)
    sslab = jnp.full((SLAB_CH * CH,), N_NODES, i32).at[pos].set(src)
    dslab = (N_NODES + slot % pad_n).at[pos].set(dst)
    sslab = sslab.reshape(SLAB_CH, CH)
    dslab = dslab.reshape(SLAB_CH, CH)
    nch = (nch.astype(i32)[:, None] * jnp.ones((1, 16), i32))
    cbase = (cbase.astype(i32)[:, None] * jnp.ones((1, 16), i32))

    zrows = jnp.zeros((CH, HF), f32)
    ones = jnp.ones((CH, HF), f32)

    bt = jnp.pad(batch.astype(i32), (0, pad_n),
                 constant_values=NUM_GRAPHS).reshape(NP, 1)

    deg = _sc_deg(ones, dslab, nch, cbase, zrows)

    agg = _sc_agg(h0, h1, sslab, dslab, nch, cbase, zrows)
    b0r = b0.reshape(1, D); g0r = g0.reshape(1, D); be0r = be0.reshape(1, D)
    h0, h1 = _dense_layer(h0, h1, agg, deg, Wr0, Wn0, b0r, g0r, be0r)

    agg = _sc_agg(h0, h1, sslab, dslab, nch, cbase, zrows)
    b1r = b1.reshape(1, D); g1r = g1.reshape(1, D); be1r = be1.reshape(1, D)
    h0, h1 = _dense_layer(h0, h1, agg, deg, Wr1, Wn1, b1r, g1r, be1r)

    agg = _sc_agg(h0, h1, sslab, dslab, nch, cbase, zrows)
    b2r = b2.reshape(1, D); g2r = g2.reshape(1, D); be2r = be2.reshape(1, D)
    h0, h1 = _dense_layer(h0, h1, agg, deg, Wr2, Wn2, b2r, g2r, be2r)

    out = _readout(h0, h1, bt, Wh1, bh1.reshape(1, 128), Wh2,
                   bh2.reshape(1, 64), Wh3, bh3.reshape(1, 128))
    return out
